# baseline (speedup 1.0000x reference)
import functools

import jax
import jax.numpy as jnp
from jax import lax
from jax.experimental import pallas as pl
from jax.experimental.pallas import tpu as pltpu
from jax.experimental.pallas import tpu_sc as plsc

B, S, D = 4096, 50, 300
V = 100000
H, C = 256, 5
DP = 384
NC, NS, L = 2, 16, 16
NW = NC * NS
BPW = B // NW
NIDX = BPW * S

WIN = list(range(0, 304, L))
ZWIN = [304, 320, 336, 352, 368]


def _pool_body(x_hbm, emb_hbm, c2_hbm, out_hbm, idx_f, idx_b0, idx_b1,
               buf0, buf1, acc_v, sem0, sem1):
    wid = lax.axis_index("s") * NC + lax.axis_index("c")
    base = wid * BPW
    pltpu.sync_copy(x_hbm.at[pl.ds(base * S, NIDX)], idx_f.at[pl.ds(0, NIDX)])

    def issue(b, idx_b, buf, sem):
        for o in (0, 16, 32, 34):
            idx_b[pl.ds(o, L)] = idx_f[pl.ds(b * S + o, L)]
        pltpu.async_copy(emb_hbm.at[idx_b, pl.ds(0, 128)],
                         buf.at[:, pl.ds(0, 128)], sem)
        pltpu.async_copy(emb_hbm.at[idx_b, pl.ds(128, 128)],
                         buf.at[:, pl.ds(128, 128)], sem)
        pltpu.async_copy(c2_hbm.at[idx_b, pl.ds(0, 128)],
                         buf.at[:, pl.ds(256, 128)], sem)

    def wait_all(idx_b, buf, sem):
        pltpu.make_async_copy(emb_hbm.at[idx_b, pl.ds(0, 128)],
                              buf.at[:, pl.ds(0, 128)], sem).wait()
        pltpu.make_async_copy(emb_hbm.at[idx_b, pl.ds(128, 128)],
                              buf.at[:, pl.ds(128, 128)], sem).wait()
        pltpu.make_async_copy(c2_hbm.at[idx_b, pl.ds(0, 128)],
                              buf.at[:, pl.ds(256, 128)], sem).wait()

    zeros = jnp.zeros((L,), jnp.float32)

    def accumulate(b, buf):
        def tbody(t, acc):
            return tuple(a + buf[t, pl.ds(o, L)] for a, o in zip(acc, WIN))

        acc = lax.fori_loop(0, S, tbody,
                            tuple(jnp.zeros((L,), jnp.float32) for _ in WIN))
        inv = jnp.float32(1.0 / S)
        for a, o in zip(acc, WIN):
            acc_v[b, pl.ds(o, L)] = a * inv
        for o in ZWIN:
            acc_v[b, pl.ds(o, L)] = zeros

    issue(0, idx_b0, buf0, sem0)
    issue(1, idx_b1, buf1, sem1)

    def loop_body(i, carry):
        b = i * 2
        wait_all(idx_b0, buf0, sem0)
        accumulate(b, buf0)

        @pl.when(b + 2 < BPW)
        def _():
            issue(b + 2, idx_b0, buf0, sem0)

        wait_all(idx_b1, buf1, sem1)
        accumulate(b + 1, buf1)

        @pl.when(b + 3 < BPW)
        def _():
            issue(b + 3, idx_b1, buf1, sem1)

        return carry

    lax.fori_loop(0, BPW // 2, loop_body, 0)
    pltpu.sync_copy(acc_v, out_hbm.at[pl.ds(base, BPW)])


_pool = functools.partial(
    pl.kernel,
    out_type=jax.ShapeDtypeStruct((B, DP), jnp.float32),
    mesh=plsc.VectorSubcoreMesh(core_axis_name="c", subcore_axis_name="s"),
    scratch_types=[
        pltpu.VMEM((NIDX + L,), jnp.int32),
        pltpu.VMEM((S,), jnp.int32),
        pltpu.VMEM((S,), jnp.int32),
        pltpu.VMEM((S, DP), jnp.float32),
        pltpu.VMEM((S, DP), jnp.float32),
        pltpu.VMEM((BPW, DP), jnp.float32),
        pltpu.SemaphoreType.DMA,
        pltpu.SemaphoreType.DMA,
    ],
    compiler_params=pltpu.CompilerParams(use_tc_tiling_on_sc=True),
)(_pool_body)


def _c2pad_body(e_ref, o_ref):
    v = e_ref[...]
    col = lax.broadcasted_iota(jnp.int32, v.shape, 1)
    o_ref[...] = jnp.where(col < D - 256, v, 0.0)


_RB = 800


def _c2pad(emb):
    return pl.pallas_call(
        _c2pad_body,
        grid=(V // _RB,),
        in_specs=[pl.BlockSpec((_RB, 128), lambda i: (i, 2))],
        out_specs=pl.BlockSpec((_RB, 128), lambda i: (i, 0)),
        out_shape=jax.ShapeDtypeStruct((V, 128), jnp.float32),
    )(emb)


def _mlp_body(p_ref, w1t_ref, b1_ref, w2t_ref, b2_ref, o_ref):
    z = jnp.dot(p_ref[...], w1t_ref[...], preferred_element_type=jnp.float32)
    z = jnp.maximum(z + b1_ref[...], 0.0)
    o_ref[...] = jnp.dot(z, w2t_ref[...], preferred_element_type=jnp.float32) + b2_ref[...]


def kernel(x, emb, W1, b1, W2, b2):
    xi = x.astype(jnp.int32)
    c2p = _c2pad(emb)
    pooled = _pool(xi.reshape(-1), emb, c2p)
    w1t = jnp.pad(W1.T, ((0, DP - D), (0, 0)))
    return pl.pallas_call(
        _mlp_body,
        out_shape=jax.ShapeDtypeStruct((B, C), jnp.float32),
    )(pooled, w1t, b1.reshape(1, H), W2.T, b2.reshape(1, C))


# speedup vs baseline: 1.0538x; 1.0538x over previous
import functools

import jax
import jax.numpy as jnp
from jax import lax
from jax.experimental import pallas as pl
from jax.experimental.pallas import tpu as pltpu
from jax.experimental.pallas import tpu_sc as plsc

B, S, D = 4096, 50, 300
V = 100000
H, C = 256, 5
DP = 384
NC, NS, L = 2, 16, 16
NW = NC * NS
BPW = B // NW
NIDX = BPW * S

WIN = list(range(0, 304, L))
ZWIN = [304, 320, 336, 352, 368]


def _pool_body(x_hbm, emb_hbm, c2_hbm, out_hbm, idx_f, idx_b0, idx_b1,
               buf0, buf1, acc_v, sem0, sem1):
    wid = lax.axis_index("s") * NC + lax.axis_index("c")
    base = wid * BPW
    pltpu.sync_copy(x_hbm.at[pl.ds(base * S, NIDX)], idx_f.at[pl.ds(0, NIDX)])

    def issue(b, idx_b, buf, sem):
        for o in (0, 16, 32, 34):
            idx_b[pl.ds(o, L)] = idx_f[pl.ds(b * S + o, L)]
        pltpu.async_copy(emb_hbm.at[idx_b, pl.ds(0, 128)],
                         buf.at[:, pl.ds(0, 128)], sem)
        pltpu.async_copy(emb_hbm.at[idx_b, pl.ds(128, 128)],
                         buf.at[:, pl.ds(128, 128)], sem)
        pltpu.async_copy(c2_hbm.at[idx_b, pl.ds(0, 128)],
                         buf.at[:, pl.ds(256, 128)], sem)

    def wait_all(idx_b, buf, sem):
        pltpu.make_async_copy(emb_hbm.at[idx_b, pl.ds(0, 128)],
                              buf.at[:, pl.ds(0, 128)], sem).wait()
        pltpu.make_async_copy(emb_hbm.at[idx_b, pl.ds(128, 128)],
                              buf.at[:, pl.ds(128, 128)], sem).wait()
        pltpu.make_async_copy(c2_hbm.at[idx_b, pl.ds(0, 128)],
                              buf.at[:, pl.ds(256, 128)], sem).wait()

    zeros = jnp.zeros((L,), jnp.float32)

    def accumulate(b, buf):
        def tbody(t, acc):
            return tuple(a + buf[t, pl.ds(o, L)] for a, o in zip(acc, WIN))

        acc = lax.fori_loop(0, S, tbody,
                            tuple(jnp.zeros((L,), jnp.float32) for _ in WIN))
        inv = jnp.float32(1.0 / S)
        for a, o in zip(acc, WIN):
            acc_v[b, pl.ds(o, L)] = a * inv
        for o in ZWIN:
            acc_v[b, pl.ds(o, L)] = zeros

    issue(0, idx_b0, buf0, sem0)
    issue(1, idx_b1, buf1, sem1)

    def loop_body(i, carry):
        b = i * 2
        wait_all(idx_b0, buf0, sem0)
        accumulate(b, buf0)

        @pl.when(b + 2 < BPW)
        def _():
            issue(b + 2, idx_b0, buf0, sem0)

        wait_all(idx_b1, buf1, sem1)
        accumulate(b + 1, buf1)

        @pl.when(b + 3 < BPW)
        def _():
            issue(b + 3, idx_b1, buf1, sem1)

        return carry

    lax.fori_loop(0, BPW // 2, loop_body, 0)
    pltpu.sync_copy(acc_v, out_hbm.at[pl.ds(base, BPW)])


_pool = functools.partial(
    pl.kernel,
    out_type=jax.ShapeDtypeStruct((B, DP), jnp.float32),
    mesh=plsc.VectorSubcoreMesh(core_axis_name="c", subcore_axis_name="s"),
    scratch_types=[
        pltpu.VMEM((NIDX + L,), jnp.int32),
        pltpu.VMEM((S,), jnp.int32),
        pltpu.VMEM((S,), jnp.int32),
        pltpu.VMEM((S, DP), jnp.float32),
        pltpu.VMEM((S, DP), jnp.float32),
        pltpu.VMEM((BPW, DP), jnp.float32),
        pltpu.SemaphoreType.DMA,
        pltpu.SemaphoreType.DMA,
    ],
    compiler_params=pltpu.CompilerParams(use_tc_tiling_on_sc=True),
)(_pool_body)


def _c2pad_body(e_ref, o_ref):
    v = e_ref[...]
    col = lax.broadcasted_iota(jnp.int32, v.shape, 1)
    o_ref[...] = jnp.where(col < D - 256, v, 0.0)


_RB = 800


def _c2pad(emb):
    return pl.pallas_call(
        _c2pad_body,
        grid=(V // _RB,),
        in_specs=[pl.BlockSpec((_RB, 128), lambda i: (i, 2))],
        out_specs=pl.BlockSpec((_RB, 128), lambda i: (i, 0)),
        out_shape=jax.ShapeDtypeStruct((V, 128), jnp.float32),
    )(emb)


def _mlp_body(p_ref, w1t_ref, b1_ref, w2t_ref, b2_ref, o_ref):
    z = jnp.dot(p_ref[...], w1t_ref[...], preferred_element_type=jnp.float32)
    z = jnp.maximum(z + b1_ref[...], 0.0)
    o_ref[...] = jnp.dot(z, w2t_ref[...], preferred_element_type=jnp.float32) + b2_ref[...]


def kernel(x, emb, W1, b1, W2, b2):
    xi = x.astype(jnp.int32)
    c2p = jnp.concatenate([emb[:, 256:], jnp.zeros((V, 128 - (D - 256)), jnp.float32)], axis=1)
    pooled = _pool(xi.reshape(-1), emb, c2p)
    w1t = jnp.pad(W1.T, ((0, DP - D), (0, 0)))
    return pl.pallas_call(
        _mlp_body,
        out_shape=jax.ShapeDtypeStruct((B, C), jnp.float32),
    )(pooled, w1t, b1.reshape(1, H), W2.T, b2.reshape(1, C))


# speedup vs baseline: 1.0544x; 1.0005x over previous
import functools

import jax
import jax.numpy as jnp
from jax import lax
from jax.experimental import pallas as pl
from jax.experimental.pallas import tpu as pltpu
from jax.experimental.pallas import tpu_sc as plsc

B, S, D = 4096, 50, 300
V = 100000
H, C = 256, 5
DP = 384
NC, NS, L = 2, 16, 16
NW = NC * NS
BPW = B // NW
NIDX = BPW * S

WIN = list(range(0, 304, L))
ZWIN = [304, 320, 336, 352, 368]


def _pool_body(x_hbm, emb_hbm, c2_hbm, out_hbm, idx_f, idx_b0, idx_b1,
               buf0, buf1, acc_v, sem0, sem1):
    wid = lax.axis_index("s") * NC + lax.axis_index("c")
    base = wid * BPW
    pltpu.sync_copy(x_hbm.at[pl.ds(base * S, NIDX)], idx_f.at[pl.ds(0, NIDX)])

    def issue(b, idx_b, buf, sem):
        for o in (0, 16, 32, 34):
            idx_b[pl.ds(o, L)] = idx_f[pl.ds(b * S + o, L)]
        pltpu.async_copy(emb_hbm.at[idx_b, pl.ds(0, 128)],
                         buf.at[:, pl.ds(0, 128)], sem)
        pltpu.async_copy(emb_hbm.at[idx_b, pl.ds(128, 128)],
                         buf.at[:, pl.ds(128, 128)], sem)
        pltpu.async_copy(c2_hbm.at[idx_b, pl.ds(0, 128)],
                         buf.at[:, pl.ds(256, 128)], sem)

    def wait_all(idx_b, buf, sem):
        pltpu.make_async_copy(emb_hbm.at[idx_b, pl.ds(0, 128)],
                              buf.at[:, pl.ds(0, 128)], sem).wait()
        pltpu.make_async_copy(emb_hbm.at[idx_b, pl.ds(128, 128)],
                              buf.at[:, pl.ds(128, 128)], sem).wait()
        pltpu.make_async_copy(c2_hbm.at[idx_b, pl.ds(0, 128)],
                              buf.at[:, pl.ds(256, 128)], sem).wait()

    zeros = jnp.zeros((L,), jnp.float32)

    def accumulate(b, buf):
        def tbody(t, acc):
            return tuple(a + buf[t, pl.ds(o, L)] for a, o in zip(acc, WIN))

        acc = lax.fori_loop(0, S, tbody,
                            tuple(jnp.zeros((L,), jnp.float32) for _ in WIN))
        inv = jnp.float32(1.0 / S)
        for a, o in zip(acc, WIN):
            acc_v[b, pl.ds(o, L)] = a * inv
        for o in ZWIN:
            acc_v[b, pl.ds(o, L)] = zeros

    issue(0, idx_b0, buf0, sem0)
    issue(1, idx_b1, buf1, sem1)

    def loop_body(i, carry):
        b = i * 2
        wait_all(idx_b0, buf0, sem0)
        accumulate(b, buf0)

        @pl.when(b + 2 < BPW)
        def _():
            issue(b + 2, idx_b0, buf0, sem0)

        wait_all(idx_b1, buf1, sem1)
        accumulate(b + 1, buf1)

        @pl.when(b + 3 < BPW)
        def _():
            issue(b + 3, idx_b1, buf1, sem1)

        return carry

    lax.fori_loop(0, BPW // 2, loop_body, 0)
    pltpu.sync_copy(acc_v, out_hbm.at[pl.ds(base, BPW)])


_pool = functools.partial(
    pl.kernel,
    out_type=jax.ShapeDtypeStruct((B, DP), jnp.float32),
    mesh=plsc.VectorSubcoreMesh(core_axis_name="c", subcore_axis_name="s"),
    scratch_types=[
        pltpu.VMEM((NIDX + L,), jnp.int32),
        pltpu.VMEM((S,), jnp.int32),
        pltpu.VMEM((S,), jnp.int32),
        pltpu.VMEM((S, DP), jnp.float32),
        pltpu.VMEM((S, DP), jnp.float32),
        pltpu.VMEM((BPW, DP), jnp.float32),
        pltpu.SemaphoreType.DMA,
        pltpu.SemaphoreType.DMA,
    ],
    compiler_params=pltpu.CompilerParams(use_tc_tiling_on_sc=True),
)(_pool_body)


def _mlp_body(p_ref, w1t_ref, b1_ref, w2t_ref, b2_ref, o_ref):
    z = jnp.dot(p_ref[...], w1t_ref[...], preferred_element_type=jnp.float32)
    z = jnp.maximum(z + b1_ref[...], 0.0)
    o_ref[...] = jnp.dot(z, w2t_ref[...], preferred_element_type=jnp.float32) + b2_ref[...]


def kernel(x, emb, W1, b1, W2, b2):
    xi = x.astype(jnp.int32)
    c2p = jnp.concatenate([emb[:, 256:], jnp.zeros((V, 128 - (D - 256)), jnp.float32)], axis=1)
    pooled = _pool(xi.reshape(-1), emb, c2p)
    w1t = jnp.pad(W1.T, ((0, DP - D), (0, 0)))
    return pl.pallas_call(
        _mlp_body,
        out_shape=jax.ShapeDtypeStruct((B, C), jnp.float32),
    )(pooled, w1t, b1.reshape(1, H), W2.T, b2.reshape(1, C))
